# baseline (speedup 1.0000x reference)
import jax
import jax.numpy as jnp
from jax import lax
from jax.experimental import pallas as pl
from jax.experimental.pallas import tpu as pltpu
from jax.experimental.pallas import tpu_sc as plsc

N = 256
NBINS = 1024
SENT = 1535
HSTRIDE = 1537
LANES = 16
ZPAD = 272
JSTRIP = 32
PLANES_PER_W = 8
NW = 32

_params = pltpu.CompilerParams(use_tc_tiling_on_sc=False,
                               needs_layout_passes=False)


def _bin16(v):
    x = v * jnp.float32(NBINS - 1)
    t = x.astype(jnp.int32)
    return jnp.where(t.astype(jnp.float32) < x, t + 1, t)


def _phase1_body(img, partials, inbuf, binbuf, hist, red):
    c = lax.axis_index("c")
    s = lax.axis_index("s")
    w = s * 2 + c
    laneoff = lax.iota(jnp.int32, LANES) * HSTRIDE
    pos1 = jnp.full((LANES,), 1, jnp.int32)
    neg1 = jnp.full((LANES,), -1, jnp.int32)
    sent_vec = jnp.full((LANES,), SENT, jnp.int32)
    zero_vec = jnp.zeros((LANES,), jnp.int32)

    nzero = (HSTRIDE * LANES + LANES - 1) // LANES

    def _zero(b, _):
        for u in range(8):
            hist[pl.ds((b * 8 + u) * LANES, LANES)] = zero_vec
        return 0

    lax.fori_loop(0, nzero // 8, _zero, 0)
    for u in range(nzero - nzero // 8 * 8):
        hist[pl.ds((nzero // 8 * 8 + u) * LANES, LANES)] = zero_vec

    def load_bin_plane(ip, slot, s_idx, j0):
        @pl.when(ip <= N - 1)
        def _():
            @pl.when(s_idx < 7)
            def _():
                pltpu.sync_copy(
                    img.at[pl.ds(ip, 1), pl.ds(j0, JSTRIP + 1), :],
                    inbuf.at[pl.ds(slot, 1)])

            @pl.when(s_idx == 7)
            def _():
                pltpu.sync_copy(
                    img.at[pl.ds(ip, 1), pl.ds(j0, JSTRIP), :],
                    inbuf.at[pl.ds(slot, 1), pl.ds(0, JSTRIP)])

            nrows = jnp.where(s_idx < 7, JSTRIP + 1, JSTRIP)

            def _binrow(r, _):
                vals = [_bin16(inbuf[slot, r, pl.ds(t * LANES, LANES)])
                        for t in range(N // LANES)]
                for t in range(N // LANES):
                    binbuf[slot, r, pl.ds(t * LANES, LANES)] = vals[t]
                binbuf[slot, r, pl.ds(N, LANES)] = sent_vec
                return 0

            lax.fori_loop(0, nrows, _binrow, 0)

            @pl.when(s_idx == 7)
            def _():
                for t in range(ZPAD // LANES):
                    binbuf[slot, JSTRIP, pl.ds(t * LANES, LANES)] = sent_vec

        @pl.when(ip > N - 1)
        def _():
            def _sentrow(r, _):
                for t in range(ZPAD // LANES):
                    binbuf[slot, r, pl.ds(t * LANES, LANES)] = sent_vec
                return 0

            lax.fori_loop(0, JSTRIP + 1, _sentrow, 0)

    def anchor_pass(pa, pb):
        def _scatter_cells(jj, k):
            a = binbuf[pa, jj, pl.ds(k, LANES)]
            az = binbuf[pa, jj, pl.ds(k + 1, LANES)]
            cc = binbuf[pa, jj + 1, pl.ds(k, LANES)]
            cz = binbuf[pa, jj + 1, pl.ds(k + 1, LANES)]
            b = binbuf[pb, jj, pl.ds(k, LANES)]
            bz = binbuf[pb, jj, pl.ds(k + 1, LANES)]
            dd = binbuf[pb, jj + 1, pl.ds(k, LANES)]
            dz = binbuf[pb, jj + 1, pl.ds(k + 1, LANES)]
            m_ab = jnp.maximum(a, b)
            m_cd = jnp.maximum(cc, dd)
            m_abz = jnp.maximum(az, bz)
            m_cdz = jnp.maximum(cz, dz)
            ey = jnp.maximum(a, cc)
            m_acz = jnp.maximum(az, cz)
            sq_z = jnp.maximum(m_ab, m_cd)
            m_z2 = jnp.maximum(m_abz, m_cdz)
            k1 = az > a
            k2 = m_abz > m_ab
            k3 = m_acz > ey
            k4 = m_z2 > sq_z
            plsc.addupdate_scatter(hist, [a + laneoff], pos1, mask=k1)
            plsc.addupdate_scatter(hist, [az + laneoff], neg1, mask=k1)
            plsc.addupdate_scatter(hist, [m_ab + laneoff], neg1, mask=k2)
            plsc.addupdate_scatter(hist, [m_abz + laneoff], pos1, mask=k2)
            plsc.addupdate_scatter(hist, [ey + laneoff], neg1, mask=k3)
            plsc.addupdate_scatter(hist, [m_acz + laneoff], pos1, mask=k3)
            plsc.addupdate_scatter(hist, [sq_z + laneoff], pos1, mask=k4)
            plsc.addupdate_scatter(hist, [m_z2 + laneoff], neg1, mask=k4)

        def _row(jj, _):
            def _col(t, _):
                k = t * (2 * LANES)
                _scatter_cells(jj, k)
                _scatter_cells(jj, k + LANES)
                return 0

            lax.fori_loop(0, N // (2 * LANES), _col, 0)
            return 0

        lax.fori_loop(0, JSTRIP, _row, 0)

    def _strip(s_idx, _):
        j0 = s_idx * JSTRIP
        load_bin_plane(w * PLANES_PER_W, jnp.int32(0), s_idx, j0)

        def _plane(pp, _):
            slot = pp & 1
            load_bin_plane(w * PLANES_PER_W + pp, slot, s_idx, j0)
            anchor_pass(1 - slot, slot)
            return 0

        lax.fori_loop(1, PLANES_PER_W + 1, _plane, 0)
        return 0

    lax.fori_loop(0, 8, _strip, 0)

    def _reduce(cb, _):
        acc = zero_vec
        for l in range(LANES):
            acc = acc + hist[pl.ds(l * HSTRIDE + cb * LANES, LANES)]
        red[pl.ds(cb * LANES, LANES)] = acc
        return 0

    lax.fori_loop(0, NBINS // LANES, _reduce, 0)
    pltpu.sync_copy(red, partials.at[w])


def _phase2_body(partials, out, buf, red):
    c = lax.axis_index("c")
    s = lax.axis_index("s")
    w = s * 2 + c

    @pl.when(w == 0)
    def _():
        pltpu.sync_copy(partials, buf)

        def _chunk(cb, _):
            acc = jnp.zeros((LANES,), jnp.int32)
            for r in range(NW):
                acc = acc + buf[r, pl.ds(cb * LANES, LANES)]
            red[pl.ds(cb * LANES, LANES)] = acc
            return 0

        lax.fori_loop(0, NBINS // LANES, _chunk, 0)

        def _csum(cb, carry):
            ch = red[pl.ds(cb * LANES, LANES)]
            red[pl.ds(cb * LANES, LANES)] = plsc.cumsum(ch) + carry
            return carry + jnp.sum(ch)

        lax.fori_loop(0, NBINS // LANES, _csum, jnp.int32(0))
        pltpu.sync_copy(red, out)


def kernel(img_arr):
    mesh = plsc.VectorSubcoreMesh(core_axis_name="c", subcore_axis_name="s")
    partials = pl.kernel(
        _phase1_body,
        out_type=jax.ShapeDtypeStruct((NW, NBINS), jnp.int32),
        mesh=mesh,
        compiler_params=_params,
        scratch_types=[
            pltpu.VMEM((2, JSTRIP + 1, N), jnp.float32),
            pltpu.VMEM((2, JSTRIP + 1, ZPAD), jnp.int32),
            pltpu.VMEM((HSTRIDE * LANES + LANES,), jnp.int32),
            pltpu.VMEM((NBINS,), jnp.int32),
        ],
    )(img_arr)
    return pl.kernel(
        _phase2_body,
        out_type=jax.ShapeDtypeStruct((NBINS,), jnp.int32),
        mesh=plsc.VectorSubcoreMesh(core_axis_name="c", subcore_axis_name="s"),
        compiler_params=_params,
        scratch_types=[
            pltpu.VMEM((NW, NBINS), jnp.int32),
            pltpu.VMEM((NBINS,), jnp.int32),
        ],
    )(partials)


# speedup vs baseline: 1.3369x; 1.3369x over previous
import jax
import jax.numpy as jnp
from jax import lax
from jax.experimental import pallas as pl
from jax.experimental.pallas import tpu as pltpu
from jax.experimental.pallas import tpu_sc as plsc

N = 256
NBINS = 1024
SENT = 1535
HSTRIDE = 1537
LANES = 16
ZPAD = 272
JSTRIP = 32
PLANES_PER_W = 8
NW = 32

_params = pltpu.CompilerParams(use_tc_tiling_on_sc=False,
                               needs_layout_passes=False)


def _bin16(v):
    x = v * jnp.float32(NBINS - 1)
    t = x.astype(jnp.int32)
    return jnp.where(t.astype(jnp.float32) < x, t + 1, t)


def _phase1_body(img, partials, inbuf, binbuf, hist, red):
    c = lax.axis_index("c")
    s = lax.axis_index("s")
    w = s * 2 + c
    laneoff = lax.iota(jnp.int32, LANES) * HSTRIDE
    pos1 = jnp.full((LANES,), 1, jnp.int32)
    neg1 = jnp.full((LANES,), -1, jnp.int32)
    sent_vec = jnp.full((LANES,), SENT, jnp.int32)
    zero_vec = jnp.zeros((LANES,), jnp.int32)

    nzero = (HSTRIDE * LANES + LANES - 1) // LANES

    def _zero(b, _):
        for u in range(8):
            hist[pl.ds((b * 8 + u) * LANES, LANES)] = zero_vec
        return 0

    lax.fori_loop(0, nzero // 8, _zero, 0)
    for u in range(nzero - nzero // 8 * 8):
        hist[pl.ds((nzero // 8 * 8 + u) * LANES, LANES)] = zero_vec

    def load_bin_plane(ip, slot, s_idx, j0):
        @pl.when(ip <= N - 1)
        def _():
            @pl.when(s_idx < 7)
            def _():
                pltpu.sync_copy(
                    img.at[pl.ds(ip, 1), pl.ds(j0, JSTRIP + 1), :],
                    inbuf.at[pl.ds(slot, 1)])

            @pl.when(s_idx == 7)
            def _():
                pltpu.sync_copy(
                    img.at[pl.ds(ip, 1), pl.ds(j0, JSTRIP), :],
                    inbuf.at[pl.ds(slot, 1), pl.ds(0, JSTRIP)])

            nrows = jnp.where(s_idx < 7, JSTRIP + 1, JSTRIP)

            def _binrow(r, _):
                vals = [_bin16(inbuf[slot, r, pl.ds(t * LANES, LANES)])
                        for t in range(N // LANES)]
                for t in range(N // LANES):
                    binbuf[slot, r, pl.ds(t * LANES, LANES)] = vals[t]
                binbuf[slot, r, pl.ds(N, LANES)] = sent_vec
                return 0

            lax.fori_loop(0, nrows, _binrow, 0)

            @pl.when(s_idx == 7)
            def _():
                for t in range(ZPAD // LANES):
                    binbuf[slot, JSTRIP, pl.ds(t * LANES, LANES)] = sent_vec

        @pl.when(ip > N - 1)
        def _():
            def _sentrow(r, _):
                for t in range(ZPAD // LANES):
                    binbuf[slot, r, pl.ds(t * LANES, LANES)] = sent_vec
                return 0

            lax.fori_loop(0, JSTRIP + 1, _sentrow, 0)

    def anchor_pass(pa, pb):
        def _scatter_cells(jj, k):
            a = binbuf[pa, jj, pl.ds(k, LANES)]
            az = binbuf[pa, jj, pl.ds(k + 1, LANES)]
            cc = binbuf[pa, jj + 1, pl.ds(k, LANES)]
            cz = binbuf[pa, jj + 1, pl.ds(k + 1, LANES)]
            b = binbuf[pb, jj, pl.ds(k, LANES)]
            bz = binbuf[pb, jj, pl.ds(k + 1, LANES)]
            dd = binbuf[pb, jj + 1, pl.ds(k, LANES)]
            dz = binbuf[pb, jj + 1, pl.ds(k + 1, LANES)]
            m_ab = jnp.maximum(a, b)
            m_cd = jnp.maximum(cc, dd)
            m_abz = jnp.maximum(az, bz)
            m_cdz = jnp.maximum(cz, dz)
            ey = jnp.maximum(a, cc)
            m_acz = jnp.maximum(az, cz)
            sq_z = jnp.maximum(m_ab, m_cd)
            m_z2 = jnp.maximum(m_abz, m_cdz)
            k1 = az > a
            k2 = m_abz > m_ab
            k3 = m_acz > ey
            k4 = m_z2 > sq_z
            plsc.addupdate_scatter(hist, [a + laneoff], pos1, mask=k1)
            plsc.addupdate_scatter(hist, [az + laneoff], neg1, mask=k1)
            plsc.addupdate_scatter(hist, [m_ab + laneoff], neg1, mask=k2)
            plsc.addupdate_scatter(hist, [m_abz + laneoff], pos1, mask=k2)
            plsc.addupdate_scatter(hist, [ey + laneoff], neg1, mask=k3)
            plsc.addupdate_scatter(hist, [m_acz + laneoff], pos1, mask=k3)
            plsc.addupdate_scatter(hist, [sq_z + laneoff], pos1, mask=k4)
            plsc.addupdate_scatter(hist, [m_z2 + laneoff], neg1, mask=k4)

        def _row(jj, _):
            @plsc.parallel_loop(0, N // LANES, unroll=2)
            def _col(t):
                _scatter_cells(jj, t * LANES)

            return 0

        lax.fori_loop(0, JSTRIP, _row, 0)

    def _strip(s_idx, _):
        j0 = s_idx * JSTRIP
        load_bin_plane(w * PLANES_PER_W, jnp.int32(0), s_idx, j0)

        def _plane(pp, _):
            slot = pp & 1
            load_bin_plane(w * PLANES_PER_W + pp, slot, s_idx, j0)
            anchor_pass(1 - slot, slot)
            return 0

        lax.fori_loop(1, PLANES_PER_W + 1, _plane, 0)
        return 0

    lax.fori_loop(0, 8, _strip, 0)

    def _reduce(cb, _):
        acc = zero_vec
        for l in range(LANES):
            acc = acc + hist[pl.ds(l * HSTRIDE + cb * LANES, LANES)]
        red[pl.ds(cb * LANES, LANES)] = acc
        return 0

    lax.fori_loop(0, NBINS // LANES, _reduce, 0)
    pltpu.sync_copy(red, partials.at[w])


def _phase2_body(partials, out, buf, red):
    c = lax.axis_index("c")
    s = lax.axis_index("s")
    w = s * 2 + c

    @pl.when(w == 0)
    def _():
        pltpu.sync_copy(partials, buf)

        def _chunk(cb, _):
            acc = jnp.zeros((LANES,), jnp.int32)
            for r in range(NW):
                acc = acc + buf[r, pl.ds(cb * LANES, LANES)]
            red[pl.ds(cb * LANES, LANES)] = acc
            return 0

        lax.fori_loop(0, NBINS // LANES, _chunk, 0)

        def _csum(cb, carry):
            ch = red[pl.ds(cb * LANES, LANES)]
            red[pl.ds(cb * LANES, LANES)] = plsc.cumsum(ch) + carry
            return carry + jnp.sum(ch)

        lax.fori_loop(0, NBINS // LANES, _csum, jnp.int32(0))
        pltpu.sync_copy(red, out)


def kernel(img_arr):
    mesh = plsc.VectorSubcoreMesh(core_axis_name="c", subcore_axis_name="s")
    partials = pl.kernel(
        _phase1_body,
        out_type=jax.ShapeDtypeStruct((NW, NBINS), jnp.int32),
        mesh=mesh,
        compiler_params=_params,
        scratch_types=[
            pltpu.VMEM((2, JSTRIP + 1, N), jnp.float32),
            pltpu.VMEM((2, JSTRIP + 1, ZPAD), jnp.int32),
            pltpu.VMEM((HSTRIDE * LANES + LANES,), jnp.int32),
            pltpu.VMEM((NBINS,), jnp.int32),
        ],
    )(img_arr)
    return pl.kernel(
        _phase2_body,
        out_type=jax.ShapeDtypeStruct((NBINS,), jnp.int32),
        mesh=plsc.VectorSubcoreMesh(core_axis_name="c", subcore_axis_name="s"),
        compiler_params=_params,
        scratch_types=[
            pltpu.VMEM((NW, NBINS), jnp.int32),
            pltpu.VMEM((NBINS,), jnp.int32),
        ],
    )(partials)


# speedup vs baseline: 1.5666x; 1.1718x over previous
import jax
import jax.numpy as jnp
from jax import lax
from jax.experimental import pallas as pl
from jax.experimental.pallas import tpu as pltpu
from jax.experimental.pallas import tpu_sc as plsc

N = 256
NBINS = 1024
SENT = 1535
HSTRIDE = 1537
LANES = 16
ZPAD = 272
JSTRIP = 32
PLANES_PER_W = 8
NW = 32

_params = pltpu.CompilerParams(use_tc_tiling_on_sc=False,
                               needs_layout_passes=False)


def _bin16(v):
    x = v * jnp.float32(NBINS - 1)
    t = x.astype(jnp.int32)
    return jnp.where(t.astype(jnp.float32) < x, t + 1, t)


def _phase1_body(img, partials, inbuf, binbuf, hist, red, dsem):
    c = lax.axis_index("c")
    s = lax.axis_index("s")
    w = s * 2 + c
    laneoff = lax.iota(jnp.int32, LANES) * HSTRIDE
    pos1 = jnp.full((LANES,), 1, jnp.int32)
    neg1 = jnp.full((LANES,), -1, jnp.int32)
    sent_vec = jnp.full((LANES,), SENT, jnp.int32)
    zero_vec = jnp.zeros((LANES,), jnp.int32)

    nzero = (HSTRIDE * LANES + LANES - 1) // LANES

    def _zero(b, _):
        for u in range(8):
            hist[pl.ds((b * 8 + u) * LANES, LANES)] = zero_vec
        return 0

    lax.fori_loop(0, nzero // 8, _zero, 0)
    for u in range(nzero - nzero // 8 * 8):
        hist[pl.ds((nzero // 8 * 8 + u) * LANES, LANES)] = zero_vec

    def _dma(ip, slot, s_idx, j0, wait):
        @pl.when(ip <= N - 1)
        def _():
            @pl.when(s_idx < 7)
            def _():
                d = pltpu.make_async_copy(
                    img.at[pl.ds(ip, 1), pl.ds(j0, JSTRIP + 1), :],
                    inbuf.at[pl.ds(slot, 1)], dsem)
                d.wait() if wait else d.start()

            @pl.when(s_idx == 7)
            def _():
                d = pltpu.make_async_copy(
                    img.at[pl.ds(ip, 1), pl.ds(j0, JSTRIP), :],
                    inbuf.at[pl.ds(slot, 1), pl.ds(0, JSTRIP)], dsem)
                d.wait() if wait else d.start()

    def bin_plane(ip, slot, s_idx):
        @pl.when(ip <= N - 1)
        def _():
            nrows = jnp.where(s_idx < 7, JSTRIP + 1, JSTRIP)

            @plsc.parallel_loop(0, nrows)
            def _binrow(r):
                vals = [_bin16(inbuf[slot, r, pl.ds(t * LANES, LANES)])
                        for t in range(N // LANES)]
                for t in range(N // LANES):
                    binbuf[slot, r, pl.ds(t * LANES, LANES)] = vals[t]
                binbuf[slot, r, pl.ds(N, LANES)] = sent_vec

            @pl.when(s_idx == 7)
            def _():
                for t in range(ZPAD // LANES):
                    binbuf[slot, JSTRIP, pl.ds(t * LANES, LANES)] = sent_vec

        @pl.when(ip > N - 1)
        def _():
            def _sentrow(r, _):
                for t in range(ZPAD // LANES):
                    binbuf[slot, r, pl.ds(t * LANES, LANES)] = sent_vec
                return 0

            lax.fori_loop(0, JSTRIP + 1, _sentrow, 0)

    def anchor_pass(pa, pb):
        def _scatter_cells(jj, k):
            a = binbuf[pa, jj, pl.ds(k, LANES)]
            az = binbuf[pa, jj, pl.ds(k + 1, LANES)]
            cc = binbuf[pa, jj + 1, pl.ds(k, LANES)]
            cz = binbuf[pa, jj + 1, pl.ds(k + 1, LANES)]
            b = binbuf[pb, jj, pl.ds(k, LANES)]
            bz = binbuf[pb, jj, pl.ds(k + 1, LANES)]
            dd = binbuf[pb, jj + 1, pl.ds(k, LANES)]
            dz = binbuf[pb, jj + 1, pl.ds(k + 1, LANES)]
            m_ab = jnp.maximum(a, b)
            m_cd = jnp.maximum(cc, dd)
            m_abz = jnp.maximum(az, bz)
            m_cdz = jnp.maximum(cz, dz)
            ey = jnp.maximum(a, cc)
            m_acz = jnp.maximum(az, cz)
            sq_z = jnp.maximum(m_ab, m_cd)
            m_z2 = jnp.maximum(m_abz, m_cdz)
            k1 = az > a
            k2 = m_abz > m_ab
            k3 = m_acz > ey
            k4 = m_z2 > sq_z
            plsc.addupdate_scatter(hist, [a + laneoff], pos1, mask=k1)
            plsc.addupdate_scatter(hist, [az + laneoff], neg1, mask=k1)
            plsc.addupdate_scatter(hist, [m_ab + laneoff], neg1, mask=k2)
            plsc.addupdate_scatter(hist, [m_abz + laneoff], pos1, mask=k2)
            plsc.addupdate_scatter(hist, [ey + laneoff], neg1, mask=k3)
            plsc.addupdate_scatter(hist, [m_acz + laneoff], pos1, mask=k3)
            plsc.addupdate_scatter(hist, [sq_z + laneoff], pos1, mask=k4)
            plsc.addupdate_scatter(hist, [m_z2 + laneoff], neg1, mask=k4)

        def _row(jj, _):
            @plsc.parallel_loop(0, N // LANES, unroll=2)
            def _col(t):
                _scatter_cells(jj, t * LANES)

            return 0

        lax.fori_loop(0, JSTRIP, _row, 0)

    def _strip(s_idx, _):
        j0 = s_idx * JSTRIP
        base = w * PLANES_PER_W
        _dma(base, jnp.int32(0), s_idx, j0, False)
        _dma(base, jnp.int32(0), s_idx, j0, True)
        _dma(base + 1, jnp.int32(1), s_idx, j0, False)
        bin_plane(base, jnp.int32(0), s_idx)

        def _plane(pp, _):
            slot = pp & 1
            _dma(base + pp, slot, s_idx, j0, True)

            @pl.when(pp < PLANES_PER_W)
            def _():
                _dma(base + pp + 1, 1 - slot, s_idx, j0, False)

            bin_plane(base + pp, slot, s_idx)
            anchor_pass(1 - slot, slot)
            return 0

        lax.fori_loop(1, PLANES_PER_W + 1, _plane, 0)
        return 0

    lax.fori_loop(0, 8, _strip, 0)

    def _reduce(cb, _):
        acc = zero_vec
        for l in range(LANES):
            acc = acc + hist[pl.ds(l * HSTRIDE + cb * LANES, LANES)]
        red[pl.ds(cb * LANES, LANES)] = acc
        return 0

    lax.fori_loop(0, NBINS // LANES, _reduce, 0)
    pltpu.sync_copy(red, partials.at[w])


def _phase2_body(partials, out, buf, red):
    c = lax.axis_index("c")
    s = lax.axis_index("s")
    w = s * 2 + c

    @pl.when(w == 0)
    def _():
        pltpu.sync_copy(partials, buf)

        def _chunk(cb, _):
            acc = jnp.zeros((LANES,), jnp.int32)
            for r in range(NW):
                acc = acc + buf[r, pl.ds(cb * LANES, LANES)]
            red[pl.ds(cb * LANES, LANES)] = acc
            return 0

        lax.fori_loop(0, NBINS // LANES, _chunk, 0)

        def _csum(cb, carry):
            ch = red[pl.ds(cb * LANES, LANES)]
            red[pl.ds(cb * LANES, LANES)] = plsc.cumsum(ch) + carry
            return carry + jnp.sum(ch)

        lax.fori_loop(0, NBINS // LANES, _csum, jnp.int32(0))
        pltpu.sync_copy(red, out)


def kernel(img_arr):
    mesh = plsc.VectorSubcoreMesh(core_axis_name="c", subcore_axis_name="s")
    partials = pl.kernel(
        _phase1_body,
        out_type=jax.ShapeDtypeStruct((NW, NBINS), jnp.int32),
        mesh=mesh,
        compiler_params=_params,
        scratch_types=[
            pltpu.VMEM((2, JSTRIP + 1, N), jnp.float32),
            pltpu.VMEM((2, JSTRIP + 1, ZPAD), jnp.int32),
            pltpu.VMEM((HSTRIDE * LANES + LANES,), jnp.int32),
            pltpu.VMEM((NBINS,), jnp.int32),
            pltpu.SemaphoreType.DMA,
        ],
    )(img_arr)
    return pl.kernel(
        _phase2_body,
        out_type=jax.ShapeDtypeStruct((NBINS,), jnp.int32),
        mesh=plsc.VectorSubcoreMesh(core_axis_name="c", subcore_axis_name="s"),
        compiler_params=_params,
        scratch_types=[
            pltpu.VMEM((NW, NBINS), jnp.int32),
            pltpu.VMEM((NBINS,), jnp.int32),
        ],
    )(partials)


# speedup vs baseline: 1.5749x; 1.0053x over previous
import jax
import jax.numpy as jnp
from jax import lax
from jax.experimental import pallas as pl
from jax.experimental.pallas import tpu as pltpu
from jax.experimental.pallas import tpu_sc as plsc

N = 256
NBINS = 1024
SENT = 1535
HSTRIDE = 1537
LANES = 16
ZPAD = 272
JSTRIP = 32
PLANES_PER_W = 8
NW = 32

_params = pltpu.CompilerParams(use_tc_tiling_on_sc=False,
                               needs_layout_passes=False)


def _bin16(v):
    x = v * jnp.float32(NBINS - 1)
    t = x.astype(jnp.int32)
    return jnp.where(t.astype(jnp.float32) < x, t + 1, t)


def _phase1_body(img, partials, inbuf, binbuf, hist, red, dsem):
    c = lax.axis_index("c")
    s = lax.axis_index("s")
    w = s * 2 + c
    laneoff = lax.iota(jnp.int32, LANES) * HSTRIDE
    pos1 = jnp.full((LANES,), 1, jnp.int32)
    neg1 = jnp.full((LANES,), -1, jnp.int32)
    sent_vec = jnp.full((LANES,), SENT, jnp.int32)
    zero_vec = jnp.zeros((LANES,), jnp.int32)

    nzero = (HSTRIDE * LANES + LANES - 1) // LANES

    def _zero(b, _):
        for u in range(8):
            hist[pl.ds((b * 8 + u) * LANES, LANES)] = zero_vec
        return 0

    lax.fori_loop(0, nzero // 8, _zero, 0)
    for u in range(nzero - nzero // 8 * 8):
        hist[pl.ds((nzero // 8 * 8 + u) * LANES, LANES)] = zero_vec

    def _dma(ip, slot, s_idx, j0, wait):
        @pl.when(ip <= N - 1)
        def _():
            @pl.when(s_idx < 7)
            def _():
                d = pltpu.make_async_copy(
                    img.at[pl.ds(ip, 1), pl.ds(j0, JSTRIP + 1), :],
                    inbuf.at[pl.ds(slot, 1)], dsem)
                d.wait() if wait else d.start()

            @pl.when(s_idx == 7)
            def _():
                d = pltpu.make_async_copy(
                    img.at[pl.ds(ip, 1), pl.ds(j0, JSTRIP), :],
                    inbuf.at[pl.ds(slot, 1), pl.ds(0, JSTRIP)], dsem)
                d.wait() if wait else d.start()

    def bin_plane(ip, slot, s_idx):
        @pl.when(ip <= N - 1)
        def _():
            nrows = jnp.where(s_idx < 7, JSTRIP + 1, JSTRIP)

            @plsc.parallel_loop(0, nrows)
            def _binrow(r):
                vals = [_bin16(inbuf[slot, r, pl.ds(t * LANES, LANES)])
                        for t in range(N // LANES)]
                for t in range(N // LANES):
                    binbuf[slot, r, pl.ds(t * LANES, LANES)] = vals[t]
                binbuf[slot, r, pl.ds(N, LANES)] = sent_vec

            @pl.when(s_idx == 7)
            def _():
                for t in range(ZPAD // LANES):
                    binbuf[slot, JSTRIP, pl.ds(t * LANES, LANES)] = sent_vec

        @pl.when(ip > N - 1)
        def _():
            def _sentrow(r, _):
                for t in range(ZPAD // LANES):
                    binbuf[slot, r, pl.ds(t * LANES, LANES)] = sent_vec
                return 0

            lax.fori_loop(0, JSTRIP + 1, _sentrow, 0)

    def anchor_pass(pa, pb):
        def _scatter_cells(jj, k):
            a = binbuf[pa, jj, pl.ds(k, LANES)]
            az = binbuf[pa, jj, pl.ds(k + 1, LANES)]
            cc = binbuf[pa, jj + 1, pl.ds(k, LANES)]
            cz = binbuf[pa, jj + 1, pl.ds(k + 1, LANES)]
            b = binbuf[pb, jj, pl.ds(k, LANES)]
            bz = binbuf[pb, jj, pl.ds(k + 1, LANES)]
            dd = binbuf[pb, jj + 1, pl.ds(k, LANES)]
            dz = binbuf[pb, jj + 1, pl.ds(k + 1, LANES)]
            m_ab = jnp.maximum(a, b)
            m_cd = jnp.maximum(cc, dd)
            m_abz = jnp.maximum(az, bz)
            m_cdz = jnp.maximum(cz, dz)
            ey = jnp.maximum(a, cc)
            m_acz = jnp.maximum(az, cz)
            sq_z = jnp.maximum(m_ab, m_cd)
            m_z2 = jnp.maximum(m_abz, m_cdz)
            k1 = az > a
            k2 = m_abz > m_ab
            k3 = m_acz > ey
            k4 = m_z2 > sq_z
            plsc.addupdate_scatter(hist, [a + laneoff], pos1, mask=k1)
            plsc.addupdate_scatter(hist, [az + laneoff], neg1, mask=k1)
            plsc.addupdate_scatter(hist, [m_ab + laneoff], neg1, mask=k2)
            plsc.addupdate_scatter(hist, [m_abz + laneoff], pos1, mask=k2)
            plsc.addupdate_scatter(hist, [ey + laneoff], neg1, mask=k3)
            plsc.addupdate_scatter(hist, [m_acz + laneoff], pos1, mask=k3)
            plsc.addupdate_scatter(hist, [sq_z + laneoff], pos1, mask=k4)
            plsc.addupdate_scatter(hist, [m_z2 + laneoff], neg1, mask=k4)

        def _row(jj, _):
            @plsc.parallel_loop(0, N // LANES, unroll=4)
            def _col(t):
                _scatter_cells(jj, t * LANES)

            return 0

        lax.fori_loop(0, JSTRIP, _row, 0)

    def _strip(s_idx, _):
        j0 = s_idx * JSTRIP
        base = w * PLANES_PER_W
        _dma(base, jnp.int32(0), s_idx, j0, False)
        _dma(base, jnp.int32(0), s_idx, j0, True)
        _dma(base + 1, jnp.int32(1), s_idx, j0, False)
        bin_plane(base, jnp.int32(0), s_idx)

        def _plane(pp, _):
            slot = pp & 1
            _dma(base + pp, slot, s_idx, j0, True)

            @pl.when(pp < PLANES_PER_W)
            def _():
                _dma(base + pp + 1, 1 - slot, s_idx, j0, False)

            bin_plane(base + pp, slot, s_idx)
            anchor_pass(1 - slot, slot)
            return 0

        lax.fori_loop(1, PLANES_PER_W + 1, _plane, 0)
        return 0

    lax.fori_loop(0, 8, _strip, 0)

    def _reduce(cb, _):
        acc = zero_vec
        for l in range(LANES):
            acc = acc + hist[pl.ds(l * HSTRIDE + cb * LANES, LANES)]
        red[pl.ds(cb * LANES, LANES)] = acc
        return 0

    lax.fori_loop(0, NBINS // LANES, _reduce, 0)
    pltpu.sync_copy(red, partials.at[w])


def _phase2_body(partials, out, buf, red):
    c = lax.axis_index("c")
    s = lax.axis_index("s")
    w = s * 2 + c

    @pl.when(w == 0)
    def _():
        pltpu.sync_copy(partials, buf)

        def _chunk(cb, _):
            acc = jnp.zeros((LANES,), jnp.int32)
            for r in range(NW):
                acc = acc + buf[r, pl.ds(cb * LANES, LANES)]
            red[pl.ds(cb * LANES, LANES)] = acc
            return 0

        lax.fori_loop(0, NBINS // LANES, _chunk, 0)

        def _csum(cb, carry):
            ch = red[pl.ds(cb * LANES, LANES)]
            red[pl.ds(cb * LANES, LANES)] = plsc.cumsum(ch) + carry
            return carry + jnp.sum(ch)

        lax.fori_loop(0, NBINS // LANES, _csum, jnp.int32(0))
        pltpu.sync_copy(red, out)


def kernel(img_arr):
    mesh = plsc.VectorSubcoreMesh(core_axis_name="c", subcore_axis_name="s")
    partials = pl.kernel(
        _phase1_body,
        out_type=jax.ShapeDtypeStruct((NW, NBINS), jnp.int32),
        mesh=mesh,
        compiler_params=_params,
        scratch_types=[
            pltpu.VMEM((2, JSTRIP + 1, N), jnp.float32),
            pltpu.VMEM((2, JSTRIP + 1, ZPAD), jnp.int32),
            pltpu.VMEM((HSTRIDE * LANES + LANES,), jnp.int32),
            pltpu.VMEM((NBINS,), jnp.int32),
            pltpu.SemaphoreType.DMA,
        ],
    )(img_arr)
    return pl.kernel(
        _phase2_body,
        out_type=jax.ShapeDtypeStruct((NBINS,), jnp.int32),
        mesh=plsc.VectorSubcoreMesh(core_axis_name="c", subcore_axis_name="s"),
        compiler_params=_params,
        scratch_types=[
            pltpu.VMEM((NW, NBINS), jnp.int32),
            pltpu.VMEM((NBINS,), jnp.int32),
        ],
    )(partials)
